# baseline (speedup 1.0000x reference)
import functools

import jax
import jax.numpy as jnp
from jax import lax
from jax.experimental import pallas as pl
import jax.experimental.pallas.tpu as pltpu
from jax.experimental.pallas import tpu_sc as plsc

N = 10000
NP = 10240
D = 128
E = 320000
NC = 2
NS = 16
NW = NC * NS
EPT = 10240
EP = NW * EPT
NPT = NP // NS

CHD = 128
NCHD = EPT // CHD

CH = 128
NCHUNK = EPT // CH
BLK = 8
NBLK = NCHUNK // BLK
HC = CH // 2

_MESH = plsc.VectorSubcoreMesh(core_axis_name="c", subcore_axis_name="s",
                               num_cores=NC, num_subcores=NS)
_SC_PARAMS = pltpu.CompilerParams(needs_layout_passes=False)
_SC_PARAMS_LINEAR = pltpu.CompilerParams(needs_layout_passes=False,
                                         use_tc_tiling_on_sc=False)
DW = D // 2


def _wid(c, s):
    return s * NC + c


@functools.partial(
    pl.kernel,
    out_type=jax.ShapeDtypeStruct((NC, NP), jnp.float32),
    mesh=_MESH,
    compiler_params=_SC_PARAMS,
    scratch_types=[
        pltpu.VMEM((NCHD, CHD), jnp.int32),
        pltpu.VMEM((EPT,), jnp.float32),
        pltpu.VMEM((NPT,), jnp.float32),
        pltpu.VMEM_SHARED((NP,), jnp.float32),
    ],
)
def _deg_kernel(dst_hbm, ew_hbm, out_hbm, dst_v, ew_v, stage_v, deg_sh):
    c = lax.axis_index("c")
    s = lax.axis_index("s")
    w = _wid(c, s)
    pltpu.sync_copy(dst_hbm.at[w], dst_v)
    pltpu.sync_copy(ew_hbm.at[pl.ds(w * EPT, EPT)], ew_v)

    def zero_body(i, _):
        stage_v[pl.ds(i * 16, 16)] = jnp.zeros((16,), jnp.float32)
        return 0
    lax.fori_loop(0, NPT // 16, zero_body, 0)
    pltpu.sync_copy(stage_v, deg_sh.at[pl.ds(s * NPT, NPT)])
    plsc.subcore_barrier()

    def chunk_body(i, _):
        pltpu.sync_copy(ew_v.at[pl.ds(i * CHD, CHD)],
                        deg_sh.at[dst_v.at[i]], add=True)
        return 0
    lax.fori_loop(0, NCHD, chunk_body, 0)
    plsc.subcore_barrier()

    pltpu.sync_copy(deg_sh.at[pl.ds(s * NPT, NPT)], stage_v)
    pltpu.sync_copy(stage_v, out_hbm.at[c, pl.ds(s * NPT, NPT)])


@functools.partial(
    pl.kernel,
    out_type=jax.ShapeDtypeStruct((NC, NP, D), jnp.float32),
    mesh=_MESH,
    compiler_params=_SC_PARAMS_LINEAR,
    scratch_types=[
        pltpu.VMEM((2, BLK, CH), jnp.int32),
        pltpu.VMEM((2, BLK, CH), jnp.int32),
        pltpu.VMEM((2, BLK * CH), jnp.float32),
        pltpu.VMEM((2, CH, DW), jnp.int32),
        pltpu.VMEM((2, HC, D), jnp.float32),
        pltpu.VMEM_SHARED((NP, D), jnp.float32),
        [pltpu.SemaphoreType.DMA] * 2,
        [pltpu.SemaphoreType.DMA] * 2,
        [pltpu.SemaphoreType.DMA] * 2,
    ],
)
def _layer_kernel(g_hbm, src_hbm, dst_hbm, ew_hbm, out_hbm,
                  src_v, dst_v, ew_v, rows_v, scat_v, acc_sh,
                  gsems, bsems, ssems):
    c = lax.axis_index("c")
    s = lax.axis_index("s")
    w = _wid(c, s)

    def zero_body(r, _):
        for j in range(D // 16):
            scat_v[0, r, pl.ds(j * 16, 16)] = jnp.zeros((16,), jnp.float32)
        return 0
    lax.fori_loop(0, HC, zero_body, 0)
    for k in range(NPT // HC):
        pltpu.sync_copy(scat_v.at[0],
                        acc_sh.at[pl.ds(s * NPT + k * HC, HC)])
    plsc.subcore_barrier()

    pltpu.async_copy(src_hbm.at[w, pl.ds(0, BLK)], src_v.at[0], bsems[0])
    pltpu.async_copy(dst_hbm.at[w, pl.ds(0, BLK)], dst_v.at[0], bsems[0])
    pltpu.async_copy(ew_hbm.at[pl.ds(w * EPT, BLK * CH)], ew_v.at[0],
                     bsems[0])

    def pair_body(k2, _):
        for pb in range(2):
            b = k2 * 2 + pb
            nb = (pb + 1) % 2
            @pl.when(b + 1 < NBLK)
            def _():
                pltpu.async_copy(src_hbm.at[w, pl.ds((b + 1) * BLK, BLK)],
                                 src_v.at[nb], bsems[nb])
                pltpu.async_copy(dst_hbm.at[w, pl.ds((b + 1) * BLK, BLK)],
                                 dst_v.at[nb], bsems[nb])
                pltpu.async_copy(
                    ew_hbm.at[pl.ds(w * EPT + (b + 1) * BLK * CH, BLK * CH)],
                    ew_v.at[nb], bsems[nb])
            pltpu.make_async_copy(src_hbm.at[w, pl.ds(b * BLK, BLK)],
                                  src_v.at[pb], bsems[pb]).wait()
            pltpu.make_async_copy(dst_hbm.at[w, pl.ds(b * BLK, BLK)],
                                  dst_v.at[pb], bsems[pb]).wait()
            pltpu.make_async_copy(
                ew_hbm.at[pl.ds(w * EPT + b * BLK * CH, BLK * CH)],
                ew_v.at[pb], bsems[pb]).wait()

            pltpu.async_copy(g_hbm.at[src_v.at[pb, 0]], rows_v.at[0],
                             gsems[0])

            def group_body(k, _):
                for j in range(2):
                    il = k * 2 + j
                    jn = (j + 1) % 2
                    if j == 0:
                        pltpu.async_copy(g_hbm.at[src_v.at[pb, il + 1]],
                                         rows_v.at[jn], gsems[jn])
                    else:
                        @pl.when(k < BLK // 2 - 1)
                        def _():
                            pltpu.async_copy(g_hbm.at[src_v.at[pb, il + 1]],
                                             rows_v.at[jn], gsems[jn])
                    pltpu.make_async_copy(g_hbm.at[src_v.at[pb, il]],
                                         rows_v.at[j], gsems[j]).wait()

                    for half in range(2):
                        def _drain():
                            pltpu.make_async_copy(
                                scat_v.at[half],
                                acc_sh.at[dst_v.at[pb, il,
                                                   pl.ds(half * HC, HC)]],
                                ssems[half]).wait()
                        if pb == 0 and j == 0:
                            @pl.when(jnp.logical_or(k2 > 0, k > 0))
                            def _():
                                _drain()
                        else:
                            _drain()

                        def scale_body(g, _):
                            nv = ew_v[pb,
                                      pl.ds(il * CH + half * HC + g * 16,
                                            16)]
                            for t in range(16):
                                n = nv[t]
                                r = half * HC + g * 16 + t
                                rs = g * 16 + t
                                for d in range(DW // 16):
                                    w16 = rows_v[j, r, pl.ds(d * 16, 16)]
                                    a = plsc.bitcast(w16 << 16, jnp.float32)
                                    bb = plsc.bitcast((w16 >> 16) << 16,
                                                      jnp.float32)
                                    scat_v[half, rs,
                                           pl.ds(d * 16, 16)] = a * n
                                    scat_v[half, rs,
                                           pl.ds(DW + d * 16, 16)] = bb * n
                            return 0
                        lax.fori_loop(0, HC // 16, scale_body, 0)
                        pltpu.async_copy(
                            scat_v.at[half],
                            acc_sh.at[dst_v.at[pb, il,
                                               pl.ds(half * HC, HC)]],
                            ssems[half], add=True)
                return 0
            lax.fori_loop(0, BLK // 2, group_body, 0)
        return 0
    lax.fori_loop(0, NBLK // 2, pair_body, 0)

    for half in range(2):
        pltpu.make_async_copy(
            scat_v.at[half],
            acc_sh.at[dst_v.at[0, 0, pl.ds(half * HC, HC)]],
            ssems[half]).wait()
    plsc.subcore_barrier()

    for k in range(NPT // HC):
        sl = pl.ds(s * NPT + k * HC, HC)
        pltpu.sync_copy(acc_sh.at[sl], scat_v.at[k % 2])
        pltpu.sync_copy(scat_v.at[k % 2], out_hbm.at[c, sl])


def _prep_body(degp_ref, dis_ref):
    p = degp_ref[...]
    deg = p[: NP // D] + p[NP // D:] + 1.0
    dis_ref[...] = lax.rsqrt(deg)


def _pack_g(g):
    gb = g.astype(jnp.bfloat16)
    lo = lax.bitcast_convert_type(gb[:, :DW], jnp.uint16).astype(jnp.uint32)
    hi = lax.bitcast_convert_type(gb[:, DW:], jnp.uint16).astype(jnp.uint32)
    return (lo | (hi << 16)).astype(jnp.int32)


def _mm1_body(x_ref, w_ref, dis_ref, o_ref, op_ref):
    h = jnp.dot(x_ref[...], w_ref[...], preferred_element_type=jnp.float32)
    g = h * dis_ref[...]
    o_ref[...] = g
    op_ref[...] = _pack_g(g)


def _mid_body(p0_ref, p1_ref, g_ref, dis_ref, b_ref, w_ref, o_ref, op_ref):
    dis = dis_ref[...]
    y = (p0_ref[...] + p1_ref[...] + g_ref[...]) * dis + b_ref[...]
    y = jnp.maximum(y, 0.0)
    h2 = jnp.dot(y, w_ref[...], preferred_element_type=jnp.float32)
    g2 = h2 * dis
    o_ref[...] = g2
    op_ref[...] = _pack_g(g2)


def _final_body(p0_ref, p1_ref, g_ref, dis_ref, b_ref, o_ref):
    o_ref[...] = ((p0_ref[...] + p1_ref[...] + g_ref[...]) * dis_ref[...]
                  + b_ref[...])


_MB = 1024
_GRID = (NP // _MB,)


def _blk(shape, imap):
    return pl.BlockSpec(shape, imap)


_FULL = _blk((_MB, D), lambda i: (i, 0))
_COL = _blk((_MB, 1), lambda i: (i, 0))
_ROW = _blk((1, D), lambda i: (0, 0))
_SQ = _blk((D, D), lambda i: (0, 0))


def _tc_prep(deg_partials):
    return pl.pallas_call(
        _prep_body,
        in_specs=[_blk((2 * NP // D, D), lambda: (0, 0))],
        out_specs=_blk((NP // D, D), lambda: (0, 0)),
        out_shape=jax.ShapeDtypeStruct((NP // D, D), jnp.float32),
    )(deg_partials)


_HALFW = _blk((_MB, DW), lambda i: (i, 0))


def _tc_mm1(x, w, dis):
    return pl.pallas_call(
        _mm1_body,
        grid=_GRID,
        in_specs=[_FULL, _SQ, _COL],
        out_specs=[_FULL, _HALFW],
        out_shape=[jax.ShapeDtypeStruct((NP, D), jnp.float32),
                   jax.ShapeDtypeStruct((NP, DW), jnp.int32)],
    )(x, w, dis)


def _tc_mid(p0, p1, g, dis, b, w):
    return pl.pallas_call(
        _mid_body,
        grid=_GRID,
        in_specs=[_FULL, _FULL, _FULL, _COL, _ROW, _SQ],
        out_specs=[_FULL, _HALFW],
        out_shape=[jax.ShapeDtypeStruct((NP, D), jnp.float32),
                   jax.ShapeDtypeStruct((NP, DW), jnp.int32)],
    )(p0, p1, g, dis, b, w)


def _tc_final(p0, p1, g, dis, b):
    return pl.pallas_call(
        _final_body,
        grid=_GRID,
        in_specs=[_FULL, _FULL, _FULL, _COL, _ROW],
        out_specs=_FULL,
        out_shape=jax.ShapeDtypeStruct((NP, D), jnp.float32),
    )(p0, p1, g, dis, b)


def kernel(x, edge_index, edge_weight, W1, b1, W2, b2):
    src = edge_index[0].astype(jnp.int32)
    dst = edge_index[1].astype(jnp.int32)
    ew = edge_weight.astype(jnp.float32)

    pad_e = EP - E
    src_p = jnp.concatenate([src, jnp.zeros((pad_e,), jnp.int32)])
    dst_p = jnp.concatenate([dst, jnp.zeros((pad_e,), jnp.int32)])
    ew_p = jnp.concatenate([ew, jnp.zeros((pad_e,), jnp.float32)])
    src3d = src_p.reshape(NW, NCHUNK, CH)
    dst3d = dst_p.reshape(NW, NCHUNK, CH)
    dst3d_deg = dst_p.reshape(NW, NCHD, CHD)
    x_p = jnp.concatenate(
        [x.astype(jnp.float32), jnp.zeros((NP - N, D), jnp.float32)])

    deg_partials = _deg_kernel(dst3d_deg, ew_p)
    dis2d = _tc_prep(deg_partials.reshape(2 * NP // D, D))
    dis = dis2d.reshape(NP, 1)

    g1, g1p = _tc_mm1(x_p, W1, dis)
    p1 = _layer_kernel(g1p, src3d, dst3d, ew_p)
    g2, g2p = _tc_mid(p1[0], p1[1], g1, dis, b1.reshape(1, D), W2)
    p2 = _layer_kernel(g2p, src3d, dst3d, ew_p)
    out = _tc_final(p2[0], p2[1], g2, dis, b2.reshape(1, D))
    return out[:N]


# speedup vs baseline: 1.1587x; 1.1587x over previous
import functools

import jax
import jax.numpy as jnp
from jax import lax
from jax.experimental import pallas as pl
import jax.experimental.pallas.tpu as pltpu
from jax.experimental.pallas import tpu_sc as plsc

N = 10000
NP = 10240
D = 128
E = 320000
NC = 2
NS = 16
NW = NC * NS
EPT = 10240
EP = NW * EPT
NPT = NP // NS

CHD = 128
NCHD = EPT // CHD

CH = 128
NCHUNK = EPT // CH
BLK = 8
NBLK = NCHUNK // BLK
HC = CH // 2

_MESH = plsc.VectorSubcoreMesh(core_axis_name="c", subcore_axis_name="s",
                               num_cores=NC, num_subcores=NS)
_SC_PARAMS = pltpu.CompilerParams(needs_layout_passes=False)
_SC_PARAMS_LINEAR = pltpu.CompilerParams(needs_layout_passes=False,
                                         use_tc_tiling_on_sc=False)
DW = D // 2


def _wid(c, s):
    return s * NC + c


@functools.partial(
    pl.kernel,
    out_type=jax.ShapeDtypeStruct((NC, NP), jnp.float32),
    mesh=_MESH,
    compiler_params=_SC_PARAMS,
    scratch_types=[
        pltpu.VMEM((NCHD, CHD), jnp.int32),
        pltpu.VMEM((EPT,), jnp.float32),
        pltpu.VMEM((NPT,), jnp.float32),
        pltpu.VMEM_SHARED((NP,), jnp.float32),
    ],
)
def _deg_kernel(dst_hbm, ew_hbm, out_hbm, dst_v, ew_v, stage_v, deg_sh):
    c = lax.axis_index("c")
    s = lax.axis_index("s")
    w = _wid(c, s)
    pltpu.sync_copy(dst_hbm.at[w], dst_v)
    pltpu.sync_copy(ew_hbm.at[pl.ds(w * EPT, EPT)], ew_v)

    def zero_body(i, _):
        stage_v[pl.ds(i * 16, 16)] = jnp.zeros((16,), jnp.float32)
        return 0
    lax.fori_loop(0, NPT // 16, zero_body, 0)
    pltpu.sync_copy(stage_v, deg_sh.at[pl.ds(s * NPT, NPT)])
    plsc.subcore_barrier()

    def chunk_body(i, _):
        pltpu.sync_copy(ew_v.at[pl.ds(i * CHD, CHD)],
                        deg_sh.at[dst_v.at[i]], add=True)
        return 0
    lax.fori_loop(0, NCHD, chunk_body, 0)
    plsc.subcore_barrier()

    pltpu.sync_copy(deg_sh.at[pl.ds(s * NPT, NPT)], stage_v)
    pltpu.sync_copy(stage_v, out_hbm.at[c, pl.ds(s * NPT, NPT)])


@functools.partial(
    pl.kernel,
    out_type=jax.ShapeDtypeStruct((NC, NP, D), jnp.float32),
    mesh=_MESH,
    compiler_params=_SC_PARAMS_LINEAR,
    scratch_types=[
        pltpu.VMEM((2, BLK, CH), jnp.int32),
        pltpu.VMEM((2, BLK, CH), jnp.int32),
        pltpu.VMEM((2, BLK * CH), jnp.float32),
        pltpu.VMEM((2, CH, DW), jnp.int32),
        pltpu.VMEM((2, HC, D), jnp.float32),
        pltpu.VMEM_SHARED((NP, D), jnp.float32),
        [pltpu.SemaphoreType.DMA] * 2,
        [pltpu.SemaphoreType.DMA] * 2,
        [pltpu.SemaphoreType.DMA] * 2,
    ],
)
def _layer_kernel(g_hbm, src_hbm, dst_hbm, ew_hbm, out_hbm,
                  src_v, dst_v, ew_v, rows_v, scat_v, acc_sh,
                  gsems, bsems, ssems):
    c = lax.axis_index("c")
    s = lax.axis_index("s")
    w = _wid(c, s)

    def zero_body(r, _):
        for j in range(D // 16):
            scat_v[0, r, pl.ds(j * 16, 16)] = jnp.zeros((16,), jnp.float32)
        return 0
    lax.fori_loop(0, HC, zero_body, 0)
    for k in range(NPT // HC):
        pltpu.sync_copy(scat_v.at[0],
                        acc_sh.at[pl.ds(s * NPT + k * HC, HC)])
    plsc.subcore_barrier()

    pltpu.async_copy(src_hbm.at[w, pl.ds(0, BLK)], src_v.at[0], bsems[0])
    pltpu.async_copy(dst_hbm.at[w, pl.ds(0, BLK)], dst_v.at[0], bsems[0])
    pltpu.async_copy(ew_hbm.at[pl.ds(w * EPT, BLK * CH)], ew_v.at[0],
                     bsems[0])

    def pair_body(k2, _):
        for pb in range(2):
            b = k2 * 2 + pb
            nb = (pb + 1) % 2
            @pl.when(b + 1 < NBLK)
            def _():
                pltpu.async_copy(src_hbm.at[w, pl.ds((b + 1) * BLK, BLK)],
                                 src_v.at[nb], bsems[nb])
                pltpu.async_copy(dst_hbm.at[w, pl.ds((b + 1) * BLK, BLK)],
                                 dst_v.at[nb], bsems[nb])
                pltpu.async_copy(
                    ew_hbm.at[pl.ds(w * EPT + (b + 1) * BLK * CH, BLK * CH)],
                    ew_v.at[nb], bsems[nb])
            pltpu.make_async_copy(src_hbm.at[w, pl.ds(b * BLK, BLK)],
                                  src_v.at[pb], bsems[pb]).wait()
            pltpu.make_async_copy(dst_hbm.at[w, pl.ds(b * BLK, BLK)],
                                  dst_v.at[pb], bsems[pb]).wait()
            pltpu.make_async_copy(
                ew_hbm.at[pl.ds(w * EPT + b * BLK * CH, BLK * CH)],
                ew_v.at[pb], bsems[pb]).wait()

            pltpu.async_copy(g_hbm.at[src_v.at[pb, 0]], rows_v.at[0],
                             gsems[0])

            def group_body(k, _):
                for j in range(2):
                    il = k * 2 + j
                    jn = (j + 1) % 2
                    if j == 0:
                        pltpu.async_copy(g_hbm.at[src_v.at[pb, il + 1]],
                                         rows_v.at[jn], gsems[jn])
                    else:
                        @pl.when(k < BLK // 2 - 1)
                        def _():
                            pltpu.async_copy(g_hbm.at[src_v.at[pb, il + 1]],
                                             rows_v.at[jn], gsems[jn])
                    pltpu.make_async_copy(g_hbm.at[src_v.at[pb, il]],
                                         rows_v.at[j], gsems[j]).wait()

                    for half in range(2):
                        def _drain():
                            pltpu.make_async_copy(
                                scat_v.at[half],
                                acc_sh.at[dst_v.at[pb, il,
                                                   pl.ds(half * HC, HC)]],
                                ssems[half]).wait()
                        if pb == 0 and j == 0:
                            @pl.when(jnp.logical_or(k2 > 0, k > 0))
                            def _():
                                _drain()
                        else:
                            _drain()

                        def scale_body(g, _):
                            nv = ew_v[pb,
                                      pl.ds(il * CH + half * HC + g * 16,
                                            16)]
                            for t in range(16):
                                n = nv[t]
                                r = half * HC + g * 16 + t
                                rs = g * 16 + t
                                ws = [rows_v[j, r, pl.ds(d * 16, 16)]
                                      for d in range(DW // 16)]
                                for d in range(DW // 16):
                                    a = plsc.bitcast(ws[d] << 16,
                                                     jnp.float32)
                                    bb = plsc.bitcast((ws[d] >> 16) << 16,
                                                      jnp.float32)
                                    scat_v[half, rs,
                                           pl.ds(d * 16, 16)] = a * n
                                    scat_v[half, rs,
                                           pl.ds(DW + d * 16, 16)] = bb * n
                            return 0
                        lax.fori_loop(0, HC // 16, scale_body, 0)
                        pltpu.async_copy(
                            scat_v.at[half],
                            acc_sh.at[dst_v.at[pb, il,
                                               pl.ds(half * HC, HC)]],
                            ssems[half], add=True)
                return 0
            lax.fori_loop(0, BLK // 2, group_body, 0)
        return 0
    lax.fori_loop(0, NBLK // 2, pair_body, 0)

    for half in range(2):
        pltpu.make_async_copy(
            scat_v.at[half],
            acc_sh.at[dst_v.at[0, 0, pl.ds(half * HC, HC)]],
            ssems[half]).wait()
    plsc.subcore_barrier()

    for k in range(NPT // HC):
        sl = pl.ds(s * NPT + k * HC, HC)
        pltpu.sync_copy(acc_sh.at[sl], scat_v.at[k % 2])
        pltpu.sync_copy(scat_v.at[k % 2], out_hbm.at[c, sl])


def _prep_body(degp_ref, dis_ref):
    p = degp_ref[...]
    deg = p[: NP // D] + p[NP // D:] + 1.0
    dis_ref[...] = lax.rsqrt(deg)


def _pack_g(g):
    gb = g.astype(jnp.bfloat16)
    lo = lax.bitcast_convert_type(gb[:, :DW], jnp.uint16).astype(jnp.uint32)
    hi = lax.bitcast_convert_type(gb[:, DW:], jnp.uint16).astype(jnp.uint32)
    return (lo | (hi << 16)).astype(jnp.int32)


def _mm1_body(x_ref, w_ref, dis_ref, o_ref, op_ref):
    h = jnp.dot(x_ref[...], w_ref[...], preferred_element_type=jnp.float32)
    g = h * dis_ref[...]
    o_ref[...] = g
    op_ref[...] = _pack_g(g)


def _mid_body(p0_ref, p1_ref, g_ref, dis_ref, b_ref, w_ref, o_ref, op_ref):
    dis = dis_ref[...]
    y = (p0_ref[...] + p1_ref[...] + g_ref[...]) * dis + b_ref[...]
    y = jnp.maximum(y, 0.0)
    h2 = jnp.dot(y, w_ref[...], preferred_element_type=jnp.float32)
    g2 = h2 * dis
    o_ref[...] = g2
    op_ref[...] = _pack_g(g2)


def _final_body(p0_ref, p1_ref, g_ref, dis_ref, b_ref, o_ref):
    o_ref[...] = ((p0_ref[...] + p1_ref[...] + g_ref[...]) * dis_ref[...]
                  + b_ref[...])


_MB = 1024
_GRID = (NP // _MB,)


def _blk(shape, imap):
    return pl.BlockSpec(shape, imap)


_FULL = _blk((_MB, D), lambda i: (i, 0))
_COL = _blk((_MB, 1), lambda i: (i, 0))
_ROW = _blk((1, D), lambda i: (0, 0))
_SQ = _blk((D, D), lambda i: (0, 0))


def _tc_prep(deg_partials):
    return pl.pallas_call(
        _prep_body,
        in_specs=[_blk((2 * NP // D, D), lambda: (0, 0))],
        out_specs=_blk((NP // D, D), lambda: (0, 0)),
        out_shape=jax.ShapeDtypeStruct((NP // D, D), jnp.float32),
    )(deg_partials)


_HALFW = _blk((_MB, DW), lambda i: (i, 0))


def _tc_mm1(x, w, dis):
    return pl.pallas_call(
        _mm1_body,
        grid=_GRID,
        in_specs=[_FULL, _SQ, _COL],
        out_specs=[_FULL, _HALFW],
        out_shape=[jax.ShapeDtypeStruct((NP, D), jnp.float32),
                   jax.ShapeDtypeStruct((NP, DW), jnp.int32)],
    )(x, w, dis)


def _tc_mid(p0, p1, g, dis, b, w):
    return pl.pallas_call(
        _mid_body,
        grid=_GRID,
        in_specs=[_FULL, _FULL, _FULL, _COL, _ROW, _SQ],
        out_specs=[_FULL, _HALFW],
        out_shape=[jax.ShapeDtypeStruct((NP, D), jnp.float32),
                   jax.ShapeDtypeStruct((NP, DW), jnp.int32)],
    )(p0, p1, g, dis, b, w)


def _tc_final(p0, p1, g, dis, b):
    return pl.pallas_call(
        _final_body,
        grid=_GRID,
        in_specs=[_FULL, _FULL, _FULL, _COL, _ROW],
        out_specs=_FULL,
        out_shape=jax.ShapeDtypeStruct((NP, D), jnp.float32),
    )(p0, p1, g, dis, b)


def kernel(x, edge_index, edge_weight, W1, b1, W2, b2):
    src = edge_index[0].astype(jnp.int32)
    dst = edge_index[1].astype(jnp.int32)
    ew = edge_weight.astype(jnp.float32)

    pad_e = EP - E
    src_p = jnp.concatenate([src, jnp.zeros((pad_e,), jnp.int32)])
    dst_p = jnp.concatenate([dst, jnp.zeros((pad_e,), jnp.int32)])
    ew_p = jnp.concatenate([ew, jnp.zeros((pad_e,), jnp.float32)])
    src3d = src_p.reshape(NW, NCHUNK, CH)
    dst3d = dst_p.reshape(NW, NCHUNK, CH)
    dst3d_deg = dst_p.reshape(NW, NCHD, CHD)
    x_p = jnp.concatenate(
        [x.astype(jnp.float32), jnp.zeros((NP - N, D), jnp.float32)])

    deg_partials = _deg_kernel(dst3d_deg, ew_p)
    dis2d = _tc_prep(deg_partials.reshape(2 * NP // D, D))
    dis = dis2d.reshape(NP, 1)

    g1, g1p = _tc_mm1(x_p, W1, dis)
    p1 = _layer_kernel(g1p, src3d, dst3d, ew_p)
    g2, g2p = _tc_mid(p1[0], p1[1], g1, dis, b1.reshape(1, D), W2)
    p2 = _layer_kernel(g2p, src3d, dst3d, ew_p)
    out = _tc_final(p2[0], p2[1], g2, dis, b2.reshape(1, D))
    return out[:N]


# speedup vs baseline: 1.2028x; 1.0381x over previous
import functools

import jax
import jax.numpy as jnp
from jax import lax
from jax.experimental import pallas as pl
import jax.experimental.pallas.tpu as pltpu
from jax.experimental.pallas import tpu_sc as plsc

N = 10000
NP = 10240
D = 128
E = 320000
NC = 2
NS = 16
NW = NC * NS
EPT = 10240
EP = NW * EPT
NPT = NP // NS

CHD = 128
NCHD = EPT // CHD

CH = 128
NCHUNK = EPT // CH
BLK = 8
NBLK = NCHUNK // BLK
HC = CH // 2

_MESH = plsc.VectorSubcoreMesh(core_axis_name="c", subcore_axis_name="s",
                               num_cores=NC, num_subcores=NS)
_SC_PARAMS = pltpu.CompilerParams(needs_layout_passes=False)
_SC_PARAMS_LINEAR = pltpu.CompilerParams(needs_layout_passes=False,
                                         use_tc_tiling_on_sc=False)
DW = D // 2


def _wid(c, s):
    return s * NC + c


@functools.partial(
    pl.kernel,
    out_type=jax.ShapeDtypeStruct((NC, NP), jnp.float32),
    mesh=_MESH,
    compiler_params=_SC_PARAMS,
    scratch_types=[
        pltpu.VMEM((NCHD, CHD), jnp.int32),
        pltpu.VMEM((EPT,), jnp.float32),
        pltpu.VMEM((NPT,), jnp.float32),
        pltpu.VMEM_SHARED((NP,), jnp.float32),
    ],
)
def _deg_kernel(dst_hbm, ew_hbm, out_hbm, dst_v, ew_v, stage_v, deg_sh):
    c = lax.axis_index("c")
    s = lax.axis_index("s")
    w = _wid(c, s)
    pltpu.sync_copy(dst_hbm.at[w], dst_v)
    pltpu.sync_copy(ew_hbm.at[pl.ds(w * EPT, EPT)], ew_v)

    def zero_body(i, _):
        stage_v[pl.ds(i * 16, 16)] = jnp.zeros((16,), jnp.float32)
        return 0
    lax.fori_loop(0, NPT // 16, zero_body, 0)
    pltpu.sync_copy(stage_v, deg_sh.at[pl.ds(s * NPT, NPT)])
    plsc.subcore_barrier()

    def chunk_body(i, _):
        pltpu.sync_copy(ew_v.at[pl.ds(i * CHD, CHD)],
                        deg_sh.at[dst_v.at[i]], add=True)
        return 0
    lax.fori_loop(0, NCHD, chunk_body, 0)
    plsc.subcore_barrier()

    pltpu.sync_copy(deg_sh.at[pl.ds(s * NPT, NPT)], stage_v)
    pltpu.sync_copy(stage_v, out_hbm.at[c, pl.ds(s * NPT, NPT)])


@functools.partial(
    pl.kernel,
    out_type=jax.ShapeDtypeStruct((NC, NP, D), jnp.float32),
    mesh=_MESH,
    compiler_params=_SC_PARAMS_LINEAR,
    scratch_types=[
        pltpu.VMEM((2, BLK, CH), jnp.int32),
        pltpu.VMEM((2, BLK, CH), jnp.int32),
        pltpu.VMEM((2, BLK * CH), jnp.float32),
        pltpu.VMEM((2, CH, DW), jnp.int32),
        pltpu.VMEM((2, HC, D), jnp.float32),
        pltpu.VMEM_SHARED((NP, D), jnp.float32),
        [pltpu.SemaphoreType.DMA] * 2,
        [pltpu.SemaphoreType.DMA] * 2,
        [pltpu.SemaphoreType.DMA] * 2,
    ],
)
def _layer_kernel(g_hbm, src_hbm, dst_hbm, ew_hbm, out_hbm,
                  src_v, dst_v, ew_v, rows_v, scat_v, acc_sh,
                  gsems, bsems, ssems):
    c = lax.axis_index("c")
    s = lax.axis_index("s")
    w = _wid(c, s)

    def zero_body(r, _):
        for j in range(D // 16):
            scat_v[0, r, pl.ds(j * 16, 16)] = jnp.zeros((16,), jnp.float32)
        return 0
    lax.fori_loop(0, HC, zero_body, 0)
    for k in range(NPT // HC):
        pltpu.sync_copy(scat_v.at[0],
                        acc_sh.at[pl.ds(s * NPT + k * HC, HC)])
    plsc.subcore_barrier()

    pltpu.async_copy(src_hbm.at[w, pl.ds(0, BLK)], src_v.at[0], bsems[0])
    pltpu.async_copy(dst_hbm.at[w, pl.ds(0, BLK)], dst_v.at[0], bsems[0])
    pltpu.async_copy(ew_hbm.at[pl.ds(w * EPT, BLK * CH)], ew_v.at[0],
                     bsems[0])

    def pair_body(k2, _):
        for pb in range(2):
            b = k2 * 2 + pb
            nb = (pb + 1) % 2
            @pl.when(b + 1 < NBLK)
            def _():
                pltpu.async_copy(src_hbm.at[w, pl.ds((b + 1) * BLK, BLK)],
                                 src_v.at[nb], bsems[nb])
                pltpu.async_copy(dst_hbm.at[w, pl.ds((b + 1) * BLK, BLK)],
                                 dst_v.at[nb], bsems[nb])
                pltpu.async_copy(
                    ew_hbm.at[pl.ds(w * EPT + (b + 1) * BLK * CH, BLK * CH)],
                    ew_v.at[nb], bsems[nb])
            pltpu.make_async_copy(src_hbm.at[w, pl.ds(b * BLK, BLK)],
                                  src_v.at[pb], bsems[pb]).wait()
            pltpu.make_async_copy(dst_hbm.at[w, pl.ds(b * BLK, BLK)],
                                  dst_v.at[pb], bsems[pb]).wait()
            pltpu.make_async_copy(
                ew_hbm.at[pl.ds(w * EPT + b * BLK * CH, BLK * CH)],
                ew_v.at[pb], bsems[pb]).wait()

            pltpu.async_copy(g_hbm.at[src_v.at[pb, 0]], rows_v.at[0],
                             gsems[0])

            def group_body(k, _):
                for j in range(2):
                    il = k * 2 + j
                    jn = (j + 1) % 2
                    if j == 0:
                        pltpu.async_copy(g_hbm.at[src_v.at[pb, il + 1]],
                                         rows_v.at[jn], gsems[jn])
                    else:
                        @pl.when(k < BLK // 2 - 1)
                        def _():
                            pltpu.async_copy(g_hbm.at[src_v.at[pb, il + 1]],
                                             rows_v.at[jn], gsems[jn])
                    pltpu.make_async_copy(g_hbm.at[src_v.at[pb, il]],
                                         rows_v.at[j], gsems[j]).wait()

                    for half in range(2):
                        def _drain():
                            pltpu.make_async_copy(
                                scat_v.at[half],
                                acc_sh.at[dst_v.at[pb, il,
                                                   pl.ds(half * HC, HC)]],
                                ssems[half]).wait()
                        if pb == 0 and j == 0:
                            @pl.when(jnp.logical_or(k2 > 0, k > 0))
                            def _():
                                _drain()
                        else:
                            _drain()

                        def scale_body(g, _):
                            nv = ew_v[pb,
                                      pl.ds(il * CH + half * HC + g * 16,
                                            16)]
                            def _row_load(t):
                                r = half * HC + g * 16 + t
                                return [rows_v[j, r, pl.ds(d * 16, 16)]
                                        for d in range(DW // 16)]
                            ws = _row_load(0)
                            for t in range(16):
                                cur = ws
                                if t + 1 < 16:
                                    ws = _row_load(t + 1)
                                n = nv[t]
                                rs = g * 16 + t
                                for d in range(DW // 16):
                                    a = plsc.bitcast(cur[d] << 16,
                                                     jnp.float32)
                                    bb = plsc.bitcast((cur[d] >> 16) << 16,
                                                      jnp.float32)
                                    scat_v[half, rs,
                                           pl.ds(d * 16, 16)] = a * n
                                    scat_v[half, rs,
                                           pl.ds(DW + d * 16, 16)] = bb * n
                            return 0
                        lax.fori_loop(0, HC // 16, scale_body, 0)
                        pltpu.async_copy(
                            scat_v.at[half],
                            acc_sh.at[dst_v.at[pb, il,
                                               pl.ds(half * HC, HC)]],
                            ssems[half], add=True)
                return 0
            lax.fori_loop(0, BLK // 2, group_body, 0)
        return 0
    lax.fori_loop(0, NBLK // 2, pair_body, 0)

    for half in range(2):
        pltpu.make_async_copy(
            scat_v.at[half],
            acc_sh.at[dst_v.at[0, 0, pl.ds(half * HC, HC)]],
            ssems[half]).wait()
    plsc.subcore_barrier()

    for k in range(NPT // HC):
        sl = pl.ds(s * NPT + k * HC, HC)
        pltpu.sync_copy(acc_sh.at[sl], scat_v.at[k % 2])
        pltpu.sync_copy(scat_v.at[k % 2], out_hbm.at[c, sl])


def _prep_body(degp_ref, dis_ref):
    p = degp_ref[...]
    deg = p[: NP // D] + p[NP // D:] + 1.0
    dis_ref[...] = lax.rsqrt(deg)


def _pack_g(g):
    gb = g.astype(jnp.bfloat16)
    lo = lax.bitcast_convert_type(gb[:, :DW], jnp.uint16).astype(jnp.uint32)
    hi = lax.bitcast_convert_type(gb[:, DW:], jnp.uint16).astype(jnp.uint32)
    return (lo | (hi << 16)).astype(jnp.int32)


def _mm1_body(x_ref, w_ref, dis_ref, o_ref, op_ref):
    h = jnp.dot(x_ref[...], w_ref[...], preferred_element_type=jnp.float32)
    g = h * dis_ref[...]
    o_ref[...] = g
    op_ref[...] = _pack_g(g)


def _mid_body(p0_ref, p1_ref, g_ref, dis_ref, b_ref, w_ref, o_ref, op_ref):
    dis = dis_ref[...]
    y = (p0_ref[...] + p1_ref[...] + g_ref[...]) * dis + b_ref[...]
    y = jnp.maximum(y, 0.0)
    h2 = jnp.dot(y, w_ref[...], preferred_element_type=jnp.float32)
    g2 = h2 * dis
    o_ref[...] = g2
    op_ref[...] = _pack_g(g2)


def _final_body(p0_ref, p1_ref, g_ref, dis_ref, b_ref, o_ref):
    o_ref[...] = ((p0_ref[...] + p1_ref[...] + g_ref[...]) * dis_ref[...]
                  + b_ref[...])


_MB = 1024
_GRID = (NP // _MB,)


def _blk(shape, imap):
    return pl.BlockSpec(shape, imap)


_FULL = _blk((_MB, D), lambda i: (i, 0))
_COL = _blk((_MB, 1), lambda i: (i, 0))
_ROW = _blk((1, D), lambda i: (0, 0))
_SQ = _blk((D, D), lambda i: (0, 0))


def _tc_prep(deg_partials):
    return pl.pallas_call(
        _prep_body,
        in_specs=[_blk((2 * NP // D, D), lambda: (0, 0))],
        out_specs=_blk((NP // D, D), lambda: (0, 0)),
        out_shape=jax.ShapeDtypeStruct((NP // D, D), jnp.float32),
    )(deg_partials)


_HALFW = _blk((_MB, DW), lambda i: (i, 0))


def _tc_mm1(x, w, dis):
    return pl.pallas_call(
        _mm1_body,
        grid=_GRID,
        in_specs=[_FULL, _SQ, _COL],
        out_specs=[_FULL, _HALFW],
        out_shape=[jax.ShapeDtypeStruct((NP, D), jnp.float32),
                   jax.ShapeDtypeStruct((NP, DW), jnp.int32)],
    )(x, w, dis)


def _tc_mid(p0, p1, g, dis, b, w):
    return pl.pallas_call(
        _mid_body,
        grid=_GRID,
        in_specs=[_FULL, _FULL, _FULL, _COL, _ROW, _SQ],
        out_specs=[_FULL, _HALFW],
        out_shape=[jax.ShapeDtypeStruct((NP, D), jnp.float32),
                   jax.ShapeDtypeStruct((NP, DW), jnp.int32)],
    )(p0, p1, g, dis, b, w)


def _tc_final(p0, p1, g, dis, b):
    return pl.pallas_call(
        _final_body,
        grid=_GRID,
        in_specs=[_FULL, _FULL, _FULL, _COL, _ROW],
        out_specs=_FULL,
        out_shape=jax.ShapeDtypeStruct((NP, D), jnp.float32),
    )(p0, p1, g, dis, b)


def kernel(x, edge_index, edge_weight, W1, b1, W2, b2):
    src = edge_index[0].astype(jnp.int32)
    dst = edge_index[1].astype(jnp.int32)
    ew = edge_weight.astype(jnp.float32)

    pad_e = EP - E
    src_p = jnp.concatenate([src, jnp.zeros((pad_e,), jnp.int32)])
    dst_p = jnp.concatenate([dst, jnp.zeros((pad_e,), jnp.int32)])
    ew_p = jnp.concatenate([ew, jnp.zeros((pad_e,), jnp.float32)])
    src3d = src_p.reshape(NW, NCHUNK, CH)
    dst3d = dst_p.reshape(NW, NCHUNK, CH)
    dst3d_deg = dst_p.reshape(NW, NCHD, CHD)
    x_p = jnp.concatenate(
        [x.astype(jnp.float32), jnp.zeros((NP - N, D), jnp.float32)])

    deg_partials = _deg_kernel(dst3d_deg, ew_p)
    dis2d = _tc_prep(deg_partials.reshape(2 * NP // D, D))
    dis = dis2d.reshape(NP, 1)

    g1, g1p = _tc_mm1(x_p, W1, dis)
    p1 = _layer_kernel(g1p, src3d, dst3d, ew_p)
    g2, g2p = _tc_mid(p1[0], p1[1], g1, dis, b1.reshape(1, D), W2)
    p2 = _layer_kernel(g2p, src3d, dst3d, ew_p)
    out = _tc_final(p2[0], p2[1], g2, dis, b2.reshape(1, D))
    return out[:N]


# speedup vs baseline: 1.2426x; 1.0331x over previous
import functools

import jax
import jax.numpy as jnp
from jax import lax
from jax.experimental import pallas as pl
import jax.experimental.pallas.tpu as pltpu
from jax.experimental.pallas import tpu_sc as plsc

N = 10000
NP = 10240
D = 128
E = 320000
NC = 2
NS = 16
NW = NC * NS
EPT = 10240
EP = NW * EPT
NPT = NP // NS

CHD = 128
NCHD = EPT // CHD

CH = 128
NCHUNK = EPT // CH
BLK = 10
NBLK = NCHUNK // BLK
HC = CH // 2

_MESH = plsc.VectorSubcoreMesh(core_axis_name="c", subcore_axis_name="s",
                               num_cores=NC, num_subcores=NS)
_SC_PARAMS = pltpu.CompilerParams(needs_layout_passes=False)
_SC_PARAMS_LINEAR = pltpu.CompilerParams(needs_layout_passes=False,
                                         use_tc_tiling_on_sc=False)
DW = D // 2


def _wid(c, s):
    return s * NC + c


@functools.partial(
    pl.kernel,
    out_type=jax.ShapeDtypeStruct((NC, NP), jnp.float32),
    mesh=_MESH,
    compiler_params=_SC_PARAMS,
    scratch_types=[
        pltpu.VMEM((NCHD, CHD), jnp.int32),
        pltpu.VMEM((EPT,), jnp.float32),
        pltpu.VMEM((NPT,), jnp.float32),
        pltpu.VMEM_SHARED((NP,), jnp.float32),
    ],
)
def _deg_kernel(dst_hbm, ew_hbm, out_hbm, dst_v, ew_v, stage_v, deg_sh):
    c = lax.axis_index("c")
    s = lax.axis_index("s")
    w = _wid(c, s)
    pltpu.sync_copy(dst_hbm.at[w], dst_v)
    pltpu.sync_copy(ew_hbm.at[pl.ds(w * EPT, EPT)], ew_v)

    def zero_body(i, _):
        stage_v[pl.ds(i * 16, 16)] = jnp.zeros((16,), jnp.float32)
        return 0
    lax.fori_loop(0, NPT // 16, zero_body, 0)
    pltpu.sync_copy(stage_v, deg_sh.at[pl.ds(s * NPT, NPT)])
    plsc.subcore_barrier()

    def chunk_body(i, _):
        pltpu.sync_copy(ew_v.at[pl.ds(i * CHD, CHD)],
                        deg_sh.at[dst_v.at[i]], add=True)
        return 0
    lax.fori_loop(0, NCHD, chunk_body, 0)
    plsc.subcore_barrier()

    pltpu.sync_copy(deg_sh.at[pl.ds(s * NPT, NPT)], stage_v)
    pltpu.sync_copy(stage_v, out_hbm.at[c, pl.ds(s * NPT, NPT)])


@functools.partial(
    pl.kernel,
    out_type=jax.ShapeDtypeStruct((NC, NP, D), jnp.float32),
    mesh=_MESH,
    compiler_params=_SC_PARAMS_LINEAR,
    scratch_types=[
        pltpu.VMEM((2, BLK, CH), jnp.int32),
        pltpu.VMEM((2, BLK, CH), jnp.int32),
        pltpu.VMEM((2, BLK * CH), jnp.float32),
        pltpu.VMEM((2, CH, DW), jnp.int32),
        pltpu.VMEM((2, HC, D), jnp.float32),
        pltpu.VMEM_SHARED((NP, D), jnp.float32),
        [pltpu.SemaphoreType.DMA] * 2,
        [pltpu.SemaphoreType.DMA] * 2,
        [pltpu.SemaphoreType.DMA] * 2,
    ],
)
def _layer_kernel(g_hbm, src_hbm, dst_hbm, ew_hbm, out_hbm,
                  src_v, dst_v, ew_v, rows_v, scat_v, acc_sh,
                  gsems, bsems, ssems):
    c = lax.axis_index("c")
    s = lax.axis_index("s")
    w = _wid(c, s)

    def zero_body(r, _):
        for j in range(D // 16):
            scat_v[0, r, pl.ds(j * 16, 16)] = jnp.zeros((16,), jnp.float32)
        return 0
    lax.fori_loop(0, HC, zero_body, 0)
    for k in range(NPT // HC):
        pltpu.sync_copy(scat_v.at[0],
                        acc_sh.at[pl.ds(s * NPT + k * HC, HC)])
    plsc.subcore_barrier()

    pltpu.async_copy(src_hbm.at[w, pl.ds(0, BLK)], src_v.at[0], bsems[0])
    pltpu.async_copy(dst_hbm.at[w, pl.ds(0, BLK)], dst_v.at[0], bsems[0])
    pltpu.async_copy(ew_hbm.at[pl.ds(w * EPT, BLK * CH)], ew_v.at[0],
                     bsems[0])

    def pair_body(k2, _):
        for pb in range(2):
            b = k2 * 2 + pb
            nb = (pb + 1) % 2
            @pl.when(b + 1 < NBLK)
            def _():
                pltpu.async_copy(src_hbm.at[w, pl.ds((b + 1) * BLK, BLK)],
                                 src_v.at[nb], bsems[nb])
                pltpu.async_copy(dst_hbm.at[w, pl.ds((b + 1) * BLK, BLK)],
                                 dst_v.at[nb], bsems[nb])
                pltpu.async_copy(
                    ew_hbm.at[pl.ds(w * EPT + (b + 1) * BLK * CH, BLK * CH)],
                    ew_v.at[nb], bsems[nb])
            pltpu.make_async_copy(src_hbm.at[w, pl.ds(b * BLK, BLK)],
                                  src_v.at[pb], bsems[pb]).wait()
            pltpu.make_async_copy(dst_hbm.at[w, pl.ds(b * BLK, BLK)],
                                  dst_v.at[pb], bsems[pb]).wait()
            pltpu.make_async_copy(
                ew_hbm.at[pl.ds(w * EPT + b * BLK * CH, BLK * CH)],
                ew_v.at[pb], bsems[pb]).wait()

            pltpu.async_copy(g_hbm.at[src_v.at[pb, 0]], rows_v.at[0],
                             gsems[0])

            def group_body(k, _):
                for j in range(2):
                    il = k * 2 + j
                    jn = (j + 1) % 2
                    if j == 0:
                        pltpu.async_copy(g_hbm.at[src_v.at[pb, il + 1]],
                                         rows_v.at[jn], gsems[jn])
                    else:
                        @pl.when(k < BLK // 2 - 1)
                        def _():
                            pltpu.async_copy(g_hbm.at[src_v.at[pb, il + 1]],
                                             rows_v.at[jn], gsems[jn])
                    pltpu.make_async_copy(g_hbm.at[src_v.at[pb, il]],
                                         rows_v.at[j], gsems[j]).wait()

                    for half in range(2):
                        def _drain():
                            pltpu.make_async_copy(
                                scat_v.at[half],
                                acc_sh.at[dst_v.at[pb, il,
                                                   pl.ds(half * HC, HC)]],
                                ssems[half]).wait()
                        if pb == 0 and j == 0:
                            @pl.when(jnp.logical_or(k2 > 0, k > 0))
                            def _():
                                _drain()
                        else:
                            _drain()

                        def scale_body(g, _):
                            nv = ew_v[pb,
                                      pl.ds(il * CH + half * HC + g * 16,
                                            16)]
                            def _row_load(t):
                                r = half * HC + g * 16 + t
                                return [rows_v[j, r, pl.ds(d * 16, 16)]
                                        for d in range(DW // 16)]
                            ws = _row_load(0)
                            for t in range(16):
                                cur = ws
                                if t + 1 < 16:
                                    ws = _row_load(t + 1)
                                n = nv[t]
                                rs = g * 16 + t
                                for d in range(DW // 16):
                                    a = plsc.bitcast(cur[d] << 16,
                                                     jnp.float32)
                                    bb = plsc.bitcast((cur[d] >> 16) << 16,
                                                      jnp.float32)
                                    scat_v[half, rs,
                                           pl.ds(d * 16, 16)] = a * n
                                    scat_v[half, rs,
                                           pl.ds(DW + d * 16, 16)] = bb * n
                            return 0
                        lax.fori_loop(0, HC // 16, scale_body, 0)
                        pltpu.async_copy(
                            scat_v.at[half],
                            acc_sh.at[dst_v.at[pb, il,
                                               pl.ds(half * HC, HC)]],
                            ssems[half], add=True)
                return 0
            lax.fori_loop(0, BLK // 2, group_body, 0)
        return 0
    lax.fori_loop(0, NBLK // 2, pair_body, 0)

    for half in range(2):
        pltpu.make_async_copy(
            scat_v.at[half],
            acc_sh.at[dst_v.at[0, 0, pl.ds(half * HC, HC)]],
            ssems[half]).wait()
    plsc.subcore_barrier()

    for k in range(NPT // HC):
        sl = pl.ds(s * NPT + k * HC, HC)
        pltpu.sync_copy(acc_sh.at[sl], scat_v.at[k % 2])
        pltpu.sync_copy(scat_v.at[k % 2], out_hbm.at[c, sl])


def _prep_body(degp_ref, dis_ref):
    p = degp_ref[...]
    deg = p[: NP // D] + p[NP // D:] + 1.0
    dis_ref[...] = lax.rsqrt(deg)


def _pack_g(g):
    gb = g.astype(jnp.bfloat16)
    lo = lax.bitcast_convert_type(gb[:, :DW], jnp.uint16).astype(jnp.uint32)
    hi = lax.bitcast_convert_type(gb[:, DW:], jnp.uint16).astype(jnp.uint32)
    return (lo | (hi << 16)).astype(jnp.int32)


def _mm1_body(x_ref, w_ref, dis_ref, o_ref, op_ref):
    h = jnp.dot(x_ref[...], w_ref[...], preferred_element_type=jnp.float32)
    g = h * dis_ref[...]
    o_ref[...] = g
    op_ref[...] = _pack_g(g)


def _mid_body(p0_ref, p1_ref, g_ref, dis_ref, b_ref, w_ref, o_ref, op_ref):
    dis = dis_ref[...]
    y = (p0_ref[...] + p1_ref[...] + g_ref[...]) * dis + b_ref[...]
    y = jnp.maximum(y, 0.0)
    h2 = jnp.dot(y, w_ref[...], preferred_element_type=jnp.float32)
    g2 = h2 * dis
    o_ref[...] = g2
    op_ref[...] = _pack_g(g2)


def _final_body(p0_ref, p1_ref, g_ref, dis_ref, b_ref, o_ref):
    o_ref[...] = ((p0_ref[...] + p1_ref[...] + g_ref[...]) * dis_ref[...]
                  + b_ref[...])


_MB = 1024
_GRID = (NP // _MB,)


def _blk(shape, imap):
    return pl.BlockSpec(shape, imap)


_FULL = _blk((_MB, D), lambda i: (i, 0))
_COL = _blk((_MB, 1), lambda i: (i, 0))
_ROW = _blk((1, D), lambda i: (0, 0))
_SQ = _blk((D, D), lambda i: (0, 0))


def _tc_prep(deg_partials):
    return pl.pallas_call(
        _prep_body,
        in_specs=[_blk((2 * NP // D, D), lambda: (0, 0))],
        out_specs=_blk((NP // D, D), lambda: (0, 0)),
        out_shape=jax.ShapeDtypeStruct((NP // D, D), jnp.float32),
    )(deg_partials)


_HALFW = _blk((_MB, DW), lambda i: (i, 0))


def _tc_mm1(x, w, dis):
    return pl.pallas_call(
        _mm1_body,
        grid=_GRID,
        in_specs=[_FULL, _SQ, _COL],
        out_specs=[_FULL, _HALFW],
        out_shape=[jax.ShapeDtypeStruct((NP, D), jnp.float32),
                   jax.ShapeDtypeStruct((NP, DW), jnp.int32)],
    )(x, w, dis)


def _tc_mid(p0, p1, g, dis, b, w):
    return pl.pallas_call(
        _mid_body,
        grid=_GRID,
        in_specs=[_FULL, _FULL, _FULL, _COL, _ROW, _SQ],
        out_specs=[_FULL, _HALFW],
        out_shape=[jax.ShapeDtypeStruct((NP, D), jnp.float32),
                   jax.ShapeDtypeStruct((NP, DW), jnp.int32)],
    )(p0, p1, g, dis, b, w)


def _tc_final(p0, p1, g, dis, b):
    return pl.pallas_call(
        _final_body,
        grid=_GRID,
        in_specs=[_FULL, _FULL, _FULL, _COL, _ROW],
        out_specs=_FULL,
        out_shape=jax.ShapeDtypeStruct((NP, D), jnp.float32),
    )(p0, p1, g, dis, b)


def kernel(x, edge_index, edge_weight, W1, b1, W2, b2):
    src = edge_index[0].astype(jnp.int32)
    dst = edge_index[1].astype(jnp.int32)
    ew = edge_weight.astype(jnp.float32)

    pad_e = EP - E
    src_p = jnp.concatenate([src, jnp.zeros((pad_e,), jnp.int32)])
    dst_p = jnp.concatenate([dst, jnp.zeros((pad_e,), jnp.int32)])
    ew_p = jnp.concatenate([ew, jnp.zeros((pad_e,), jnp.float32)])
    src3d = src_p.reshape(NW, NCHUNK, CH)
    dst3d = dst_p.reshape(NW, NCHUNK, CH)
    dst3d_deg = dst_p.reshape(NW, NCHD, CHD)
    x_p = jnp.concatenate(
        [x.astype(jnp.float32), jnp.zeros((NP - N, D), jnp.float32)])

    deg_partials = _deg_kernel(dst3d_deg, ew_p)
    dis2d = _tc_prep(deg_partials.reshape(2 * NP // D, D))
    dis = dis2d.reshape(NP, 1)

    g1, g1p = _tc_mm1(x_p, W1, dis)
    p1 = _layer_kernel(g1p, src3d, dst3d, ew_p)
    g2, g2p = _tc_mid(p1[0], p1[1], g1, dis, b1.reshape(1, D), W2)
    p2 = _layer_kernel(g2p, src3d, dst3d, ew_p)
    out = _tc_final(p2[0], p2[1], g2, dis, b2.reshape(1, D))
    return out[:N]


# speedup vs baseline: 1.3013x; 1.0472x over previous
import functools

import jax
import jax.numpy as jnp
from jax import lax
from jax.experimental import pallas as pl
import jax.experimental.pallas.tpu as pltpu
from jax.experimental.pallas import tpu_sc as plsc

N = 10000
NP = 10240
D = 128
E = 320000
NC = 2
NS = 16
NW = NC * NS
EPT = 10240
EP = NW * EPT
NPT = NP // NS

CHD = 128
NCHD = EPT // CHD

CH = 128
NCHUNK = EPT // CH
BLK = 20
NBLK = NCHUNK // BLK
HC = CH // 2

_MESH = plsc.VectorSubcoreMesh(core_axis_name="c", subcore_axis_name="s",
                               num_cores=NC, num_subcores=NS)
_SC_PARAMS = pltpu.CompilerParams(needs_layout_passes=False)
_SC_PARAMS_LINEAR = pltpu.CompilerParams(needs_layout_passes=False,
                                         use_tc_tiling_on_sc=False)
DW = D // 2


def _wid(c, s):
    return s * NC + c


@functools.partial(
    pl.kernel,
    out_type=jax.ShapeDtypeStruct((NC, NP), jnp.float32),
    mesh=_MESH,
    compiler_params=_SC_PARAMS,
    scratch_types=[
        pltpu.VMEM((NCHD, CHD), jnp.int32),
        pltpu.VMEM((EPT,), jnp.float32),
        pltpu.VMEM((NPT,), jnp.float32),
        pltpu.VMEM_SHARED((NP,), jnp.float32),
    ],
)
def _deg_kernel(dst_hbm, ew_hbm, out_hbm, dst_v, ew_v, stage_v, deg_sh):
    c = lax.axis_index("c")
    s = lax.axis_index("s")
    w = _wid(c, s)
    pltpu.sync_copy(dst_hbm.at[w], dst_v)
    pltpu.sync_copy(ew_hbm.at[pl.ds(w * EPT, EPT)], ew_v)

    def zero_body(i, _):
        stage_v[pl.ds(i * 16, 16)] = jnp.zeros((16,), jnp.float32)
        return 0
    lax.fori_loop(0, NPT // 16, zero_body, 0)
    pltpu.sync_copy(stage_v, deg_sh.at[pl.ds(s * NPT, NPT)])
    plsc.subcore_barrier()

    def chunk_body(i, _):
        pltpu.sync_copy(ew_v.at[pl.ds(i * CHD, CHD)],
                        deg_sh.at[dst_v.at[i]], add=True)
        return 0
    lax.fori_loop(0, NCHD, chunk_body, 0)
    plsc.subcore_barrier()

    pltpu.sync_copy(deg_sh.at[pl.ds(s * NPT, NPT)], stage_v)
    pltpu.sync_copy(stage_v, out_hbm.at[c, pl.ds(s * NPT, NPT)])


@functools.partial(
    pl.kernel,
    out_type=jax.ShapeDtypeStruct((NC, NP, D), jnp.float32),
    mesh=_MESH,
    compiler_params=_SC_PARAMS_LINEAR,
    scratch_types=[
        pltpu.VMEM((2, BLK, CH), jnp.int32),
        pltpu.VMEM((2, BLK, CH), jnp.int32),
        pltpu.VMEM((2, BLK * CH), jnp.float32),
        pltpu.VMEM((2, CH, DW), jnp.int32),
        pltpu.VMEM((2, HC, D), jnp.float32),
        pltpu.VMEM_SHARED((NP, D), jnp.float32),
        [pltpu.SemaphoreType.DMA] * 2,
        [pltpu.SemaphoreType.DMA] * 2,
        [pltpu.SemaphoreType.DMA] * 2,
    ],
)
def _layer_kernel(g_hbm, src_hbm, dst_hbm, ew_hbm, out_hbm,
                  src_v, dst_v, ew_v, rows_v, scat_v, acc_sh,
                  gsems, bsems, ssems):
    c = lax.axis_index("c")
    s = lax.axis_index("s")
    w = _wid(c, s)

    def zero_body(r, _):
        for j in range(D // 16):
            scat_v[0, r, pl.ds(j * 16, 16)] = jnp.zeros((16,), jnp.float32)
        return 0
    lax.fori_loop(0, HC, zero_body, 0)
    for k in range(NPT // HC):
        pltpu.sync_copy(scat_v.at[0],
                        acc_sh.at[pl.ds(s * NPT + k * HC, HC)])
    plsc.subcore_barrier()

    pltpu.async_copy(src_hbm.at[w, pl.ds(0, BLK)], src_v.at[0], bsems[0])
    pltpu.async_copy(dst_hbm.at[w, pl.ds(0, BLK)], dst_v.at[0], bsems[0])
    pltpu.async_copy(ew_hbm.at[pl.ds(w * EPT, BLK * CH)], ew_v.at[0],
                     bsems[0])

    def pair_body(k2, _):
        for pb in range(2):
            b = k2 * 2 + pb
            nb = (pb + 1) % 2
            @pl.when(b + 1 < NBLK)
            def _():
                pltpu.async_copy(src_hbm.at[w, pl.ds((b + 1) * BLK, BLK)],
                                 src_v.at[nb], bsems[nb])
                pltpu.async_copy(dst_hbm.at[w, pl.ds((b + 1) * BLK, BLK)],
                                 dst_v.at[nb], bsems[nb])
                pltpu.async_copy(
                    ew_hbm.at[pl.ds(w * EPT + (b + 1) * BLK * CH, BLK * CH)],
                    ew_v.at[nb], bsems[nb])
            pltpu.make_async_copy(src_hbm.at[w, pl.ds(b * BLK, BLK)],
                                  src_v.at[pb], bsems[pb]).wait()
            pltpu.make_async_copy(dst_hbm.at[w, pl.ds(b * BLK, BLK)],
                                  dst_v.at[pb], bsems[pb]).wait()
            pltpu.make_async_copy(
                ew_hbm.at[pl.ds(w * EPT + b * BLK * CH, BLK * CH)],
                ew_v.at[pb], bsems[pb]).wait()

            pltpu.async_copy(g_hbm.at[src_v.at[pb, 0]], rows_v.at[0],
                             gsems[0])

            def group_body(k, _):
                for j in range(2):
                    il = k * 2 + j
                    jn = (j + 1) % 2
                    if j == 0:
                        pltpu.async_copy(g_hbm.at[src_v.at[pb, il + 1]],
                                         rows_v.at[jn], gsems[jn])
                    else:
                        @pl.when(k < BLK // 2 - 1)
                        def _():
                            pltpu.async_copy(g_hbm.at[src_v.at[pb, il + 1]],
                                             rows_v.at[jn], gsems[jn])
                    pltpu.make_async_copy(g_hbm.at[src_v.at[pb, il]],
                                         rows_v.at[j], gsems[j]).wait()

                    for half in range(2):
                        def _drain():
                            pltpu.make_async_copy(
                                scat_v.at[half],
                                acc_sh.at[dst_v.at[pb, il,
                                                   pl.ds(half * HC, HC)]],
                                ssems[half]).wait()
                        if pb == 0 and j == 0:
                            @pl.when(jnp.logical_or(k2 > 0, k > 0))
                            def _():
                                _drain()
                        else:
                            _drain()

                        def scale_body(g, _):
                            nv = ew_v[pb,
                                      pl.ds(il * CH + half * HC + g * 16,
                                            16)]
                            def _row_load(t):
                                r = half * HC + g * 16 + t
                                return [rows_v[j, r, pl.ds(d * 16, 16)]
                                        for d in range(DW // 16)]
                            ws = _row_load(0)
                            for t in range(16):
                                cur = ws
                                if t + 1 < 16:
                                    ws = _row_load(t + 1)
                                n = nv[t]
                                rs = g * 16 + t
                                for d in range(DW // 16):
                                    a = plsc.bitcast(cur[d] << 16,
                                                     jnp.float32)
                                    bb = plsc.bitcast((cur[d] >> 16) << 16,
                                                      jnp.float32)
                                    scat_v[half, rs,
                                           pl.ds(d * 16, 16)] = a * n
                                    scat_v[half, rs,
                                           pl.ds(DW + d * 16, 16)] = bb * n
                            return 0
                        lax.fori_loop(0, HC // 16, scale_body, 0)
                        pltpu.async_copy(
                            scat_v.at[half],
                            acc_sh.at[dst_v.at[pb, il,
                                               pl.ds(half * HC, HC)]],
                            ssems[half], add=True)
                return 0
            lax.fori_loop(0, BLK // 2, group_body, 0)
        return 0
    lax.fori_loop(0, NBLK // 2, pair_body, 0)

    for half in range(2):
        pltpu.make_async_copy(
            scat_v.at[half],
            acc_sh.at[dst_v.at[0, 0, pl.ds(half * HC, HC)]],
            ssems[half]).wait()
    plsc.subcore_barrier()

    for k in range(NPT // HC):
        sl = pl.ds(s * NPT + k * HC, HC)
        pltpu.sync_copy(acc_sh.at[sl], scat_v.at[k % 2])
        pltpu.sync_copy(scat_v.at[k % 2], out_hbm.at[c, sl])


def _prep_body(degp_ref, dis_ref):
    p = degp_ref[...]
    deg = p[: NP // D] + p[NP // D:] + 1.0
    dis_ref[...] = lax.rsqrt(deg)


def _pack_g(g):
    gb = g.astype(jnp.bfloat16)
    lo = lax.bitcast_convert_type(gb[:, :DW], jnp.uint16).astype(jnp.uint32)
    hi = lax.bitcast_convert_type(gb[:, DW:], jnp.uint16).astype(jnp.uint32)
    return (lo | (hi << 16)).astype(jnp.int32)


def _mm1_body(x_ref, w_ref, dis_ref, o_ref, op_ref):
    h = jnp.dot(x_ref[...], w_ref[...], preferred_element_type=jnp.float32)
    g = h * dis_ref[...]
    o_ref[...] = g
    op_ref[...] = _pack_g(g)


def _mid_body(p0_ref, p1_ref, g_ref, dis_ref, b_ref, w_ref, o_ref, op_ref):
    dis = dis_ref[...]
    y = (p0_ref[...] + p1_ref[...] + g_ref[...]) * dis + b_ref[...]
    y = jnp.maximum(y, 0.0)
    h2 = jnp.dot(y, w_ref[...], preferred_element_type=jnp.float32)
    g2 = h2 * dis
    o_ref[...] = g2
    op_ref[...] = _pack_g(g2)


def _final_body(p0_ref, p1_ref, g_ref, dis_ref, b_ref, o_ref):
    o_ref[...] = ((p0_ref[...] + p1_ref[...] + g_ref[...]) * dis_ref[...]
                  + b_ref[...])


_MB = 1024
_GRID = (NP // _MB,)


def _blk(shape, imap):
    return pl.BlockSpec(shape, imap)


_FULL = _blk((_MB, D), lambda i: (i, 0))
_COL = _blk((_MB, 1), lambda i: (i, 0))
_ROW = _blk((1, D), lambda i: (0, 0))
_SQ = _blk((D, D), lambda i: (0, 0))


def _tc_prep(deg_partials):
    return pl.pallas_call(
        _prep_body,
        in_specs=[_blk((2 * NP // D, D), lambda: (0, 0))],
        out_specs=_blk((NP // D, D), lambda: (0, 0)),
        out_shape=jax.ShapeDtypeStruct((NP // D, D), jnp.float32),
    )(deg_partials)


_HALFW = _blk((_MB, DW), lambda i: (i, 0))


def _tc_mm1(x, w, dis):
    return pl.pallas_call(
        _mm1_body,
        grid=_GRID,
        in_specs=[_FULL, _SQ, _COL],
        out_specs=[_FULL, _HALFW],
        out_shape=[jax.ShapeDtypeStruct((NP, D), jnp.float32),
                   jax.ShapeDtypeStruct((NP, DW), jnp.int32)],
    )(x, w, dis)


def _tc_mid(p0, p1, g, dis, b, w):
    return pl.pallas_call(
        _mid_body,
        grid=_GRID,
        in_specs=[_FULL, _FULL, _FULL, _COL, _ROW, _SQ],
        out_specs=[_FULL, _HALFW],
        out_shape=[jax.ShapeDtypeStruct((NP, D), jnp.float32),
                   jax.ShapeDtypeStruct((NP, DW), jnp.int32)],
    )(p0, p1, g, dis, b, w)


def _tc_final(p0, p1, g, dis, b):
    return pl.pallas_call(
        _final_body,
        grid=_GRID,
        in_specs=[_FULL, _FULL, _FULL, _COL, _ROW],
        out_specs=_FULL,
        out_shape=jax.ShapeDtypeStruct((NP, D), jnp.float32),
    )(p0, p1, g, dis, b)


def kernel(x, edge_index, edge_weight, W1, b1, W2, b2):
    src = edge_index[0].astype(jnp.int32)
    dst = edge_index[1].astype(jnp.int32)
    ew = edge_weight.astype(jnp.float32)

    pad_e = EP - E
    src_p = jnp.concatenate([src, jnp.zeros((pad_e,), jnp.int32)])
    dst_p = jnp.concatenate([dst, jnp.zeros((pad_e,), jnp.int32)])
    ew_p = jnp.concatenate([ew, jnp.zeros((pad_e,), jnp.float32)])
    src3d = src_p.reshape(NW, NCHUNK, CH)
    dst3d = dst_p.reshape(NW, NCHUNK, CH)
    dst3d_deg = dst_p.reshape(NW, NCHD, CHD)
    x_p = jnp.concatenate(
        [x.astype(jnp.float32), jnp.zeros((NP - N, D), jnp.float32)])

    deg_partials = _deg_kernel(dst3d_deg, ew_p)
    dis2d = _tc_prep(deg_partials.reshape(2 * NP // D, D))
    dis = dis2d.reshape(NP, 1)

    g1, g1p = _tc_mm1(x_p, W1, dis)
    p1 = _layer_kernel(g1p, src3d, dst3d, ew_p)
    g2, g2p = _tc_mid(p1[0], p1[1], g1, dis, b1.reshape(1, D), W2)
    p2 = _layer_kernel(g2p, src3d, dst3d, ew_p)
    out = _tc_final(p2[0], p2[1], g2, dis, b2.reshape(1, D))
    return out[:N]


# speedup vs baseline: 1.4217x; 1.0925x over previous
import functools

import jax
import jax.numpy as jnp
from jax import lax
from jax.experimental import pallas as pl
import jax.experimental.pallas.tpu as pltpu
from jax.experimental.pallas import tpu_sc as plsc

N = 10000
NP = 10240
D = 128
E = 320000
NC = 2
NS = 16
NW = NC * NS
EPT = 10240
EP = NW * EPT
NPT = NP // NS

CHD = 128
NCHD = EPT // CHD

CH = 128
NCHUNK = EPT // CH
BLK = 20
NBLK = NCHUNK // BLK
HC = CH // 2

_MESH = plsc.VectorSubcoreMesh(core_axis_name="c", subcore_axis_name="s",
                               num_cores=NC, num_subcores=NS)
_SC_PARAMS = pltpu.CompilerParams(needs_layout_passes=False)
_SC_PARAMS_LINEAR = pltpu.CompilerParams(needs_layout_passes=False,
                                         use_tc_tiling_on_sc=False)
DW = D // 2


def _wid(c, s):
    return s * NC + c


@functools.partial(
    pl.kernel,
    out_type=jax.ShapeDtypeStruct((NC, NP), jnp.float32),
    mesh=_MESH,
    compiler_params=_SC_PARAMS,
    scratch_types=[
        pltpu.VMEM((NCHD, CHD), jnp.int32),
        pltpu.VMEM((EPT,), jnp.float32),
        pltpu.VMEM((NPT,), jnp.float32),
        pltpu.VMEM_SHARED((NP,), jnp.float32),
    ],
)
def _deg_kernel(dst_hbm, ew_hbm, out_hbm, dst_v, ew_v, stage_v, deg_sh):
    c = lax.axis_index("c")
    s = lax.axis_index("s")
    w = _wid(c, s)
    pltpu.sync_copy(dst_hbm.at[w], dst_v)
    pltpu.sync_copy(ew_hbm.at[pl.ds(w * EPT, EPT)], ew_v)

    def zero_body(i, _):
        stage_v[pl.ds(i * 16, 16)] = jnp.zeros((16,), jnp.float32)
        return 0
    lax.fori_loop(0, NPT // 16, zero_body, 0)
    pltpu.sync_copy(stage_v, deg_sh.at[pl.ds(s * NPT, NPT)])
    plsc.subcore_barrier()

    def chunk_body(i, _):
        pltpu.sync_copy(ew_v.at[pl.ds(i * CHD, CHD)],
                        deg_sh.at[dst_v.at[i]], add=True)
        return 0
    lax.fori_loop(0, NCHD, chunk_body, 0)
    plsc.subcore_barrier()

    pltpu.sync_copy(deg_sh.at[pl.ds(s * NPT, NPT)], stage_v)
    pltpu.sync_copy(stage_v, out_hbm.at[c, pl.ds(s * NPT, NPT)])


@functools.partial(
    pl.kernel,
    out_type=jax.ShapeDtypeStruct((NC, NP, D), jnp.float32),
    mesh=_MESH,
    compiler_params=_SC_PARAMS_LINEAR,
    scratch_types=[
        pltpu.VMEM((2, BLK, CH), jnp.int32),
        pltpu.VMEM((2, BLK, CH), jnp.int32),
        pltpu.VMEM((2, BLK * CH), jnp.float32),
        pltpu.VMEM((2, CH, DW), jnp.int32),
        pltpu.VMEM((2, HC, D), jnp.float32),
        pltpu.VMEM_SHARED((NP, D), jnp.float32),
        [pltpu.SemaphoreType.DMA] * 2,
        [pltpu.SemaphoreType.DMA] * 2,
        [pltpu.SemaphoreType.DMA] * 2,
    ],
)
def _layer_kernel(g_hbm, src_hbm, dst_hbm, ew_hbm, out_hbm,
                  src_v, dst_v, ew_v, rows_v, scat_v, acc_sh,
                  gsems, bsems, ssems):
    c = lax.axis_index("c")
    s = lax.axis_index("s")
    w = _wid(c, s)
    gc_hbm = g_hbm.at[c]

    def zero_body(r, _):
        for j in range(D // 16):
            scat_v[0, r, pl.ds(j * 16, 16)] = jnp.zeros((16,), jnp.float32)
        return 0
    lax.fori_loop(0, HC, zero_body, 0)
    for k in range(NPT // HC):
        pltpu.sync_copy(scat_v.at[0],
                        acc_sh.at[pl.ds(s * NPT + k * HC, HC)])
    plsc.subcore_barrier()

    pltpu.async_copy(src_hbm.at[w, pl.ds(0, BLK)], src_v.at[0], bsems[0])
    pltpu.async_copy(dst_hbm.at[w, pl.ds(0, BLK)], dst_v.at[0], bsems[0])
    pltpu.async_copy(ew_hbm.at[pl.ds(w * EPT, BLK * CH)], ew_v.at[0],
                     bsems[0])

    def pair_body(k2, _):
        for pb in range(2):
            b = k2 * 2 + pb
            nb = (pb + 1) % 2
            @pl.when(b + 1 < NBLK)
            def _():
                pltpu.async_copy(src_hbm.at[w, pl.ds((b + 1) * BLK, BLK)],
                                 src_v.at[nb], bsems[nb])
                pltpu.async_copy(dst_hbm.at[w, pl.ds((b + 1) * BLK, BLK)],
                                 dst_v.at[nb], bsems[nb])
                pltpu.async_copy(
                    ew_hbm.at[pl.ds(w * EPT + (b + 1) * BLK * CH, BLK * CH)],
                    ew_v.at[nb], bsems[nb])
            pltpu.make_async_copy(src_hbm.at[w, pl.ds(b * BLK, BLK)],
                                  src_v.at[pb], bsems[pb]).wait()
            pltpu.make_async_copy(dst_hbm.at[w, pl.ds(b * BLK, BLK)],
                                  dst_v.at[pb], bsems[pb]).wait()
            pltpu.make_async_copy(
                ew_hbm.at[pl.ds(w * EPT + b * BLK * CH, BLK * CH)],
                ew_v.at[pb], bsems[pb]).wait()

            pltpu.async_copy(gc_hbm.at[src_v.at[pb, 0]], rows_v.at[0],
                             gsems[0])

            def group_body(k, _):
                for j in range(2):
                    il = k * 2 + j
                    jn = (j + 1) % 2
                    if j == 0:
                        pltpu.async_copy(gc_hbm.at[src_v.at[pb, il + 1]],
                                         rows_v.at[jn], gsems[jn])
                    else:
                        @pl.when(k < BLK // 2 - 1)
                        def _():
                            pltpu.async_copy(gc_hbm.at[src_v.at[pb, il + 1]],
                                             rows_v.at[jn], gsems[jn])
                    pltpu.make_async_copy(gc_hbm.at[src_v.at[pb, il]],
                                         rows_v.at[j], gsems[j]).wait()

                    for half in range(2):
                        def _drain():
                            pltpu.make_async_copy(
                                scat_v.at[half],
                                acc_sh.at[dst_v.at[pb, il,
                                                   pl.ds(half * HC, HC)]],
                                ssems[half]).wait()
                        if pb == 0 and j == 0:
                            @pl.when(jnp.logical_or(k2 > 0, k > 0))
                            def _():
                                _drain()
                        else:
                            _drain()

                        def scale_body(g, _):
                            nv = ew_v[pb,
                                      pl.ds(il * CH + half * HC + g * 16,
                                            16)]
                            def _row_load(t):
                                r = half * HC + g * 16 + t
                                return [rows_v[j, r, pl.ds(d * 16, 16)]
                                        for d in range(DW // 16)]
                            ws = _row_load(0)
                            for t in range(16):
                                cur = ws
                                if t + 1 < 16:
                                    ws = _row_load(t + 1)
                                n = nv[t]
                                rs = g * 16 + t
                                for d in range(DW // 16):
                                    a = plsc.bitcast(cur[d] << 16,
                                                     jnp.float32)
                                    bb = plsc.bitcast((cur[d] >> 16) << 16,
                                                      jnp.float32)
                                    scat_v[half, rs,
                                           pl.ds(d * 16, 16)] = a * n
                                    scat_v[half, rs,
                                           pl.ds(DW + d * 16, 16)] = bb * n
                            return 0
                        lax.fori_loop(0, HC // 16, scale_body, 0)
                        pltpu.async_copy(
                            scat_v.at[half],
                            acc_sh.at[dst_v.at[pb, il,
                                               pl.ds(half * HC, HC)]],
                            ssems[half], add=True)
                return 0
            lax.fori_loop(0, BLK // 2, group_body, 0)
        return 0
    lax.fori_loop(0, NBLK // 2, pair_body, 0)

    for half in range(2):
        pltpu.make_async_copy(
            scat_v.at[half],
            acc_sh.at[dst_v.at[0, 0, pl.ds(half * HC, HC)]],
            ssems[half]).wait()
    plsc.subcore_barrier()

    for k in range(NPT // HC):
        sl = pl.ds(s * NPT + k * HC, HC)
        pltpu.sync_copy(acc_sh.at[sl], scat_v.at[k % 2])
        pltpu.sync_copy(scat_v.at[k % 2], out_hbm.at[c, sl])


def _prep_body(degp_ref, dis_ref):
    p = degp_ref[...]
    deg = p[: NP // D] + p[NP // D:] + 1.0
    dis_ref[...] = lax.rsqrt(deg)


def _pack_g(g):
    gb = g.astype(jnp.bfloat16)
    lo = lax.bitcast_convert_type(gb[:, :DW], jnp.uint16).astype(jnp.uint32)
    hi = lax.bitcast_convert_type(gb[:, DW:], jnp.uint16).astype(jnp.uint32)
    return (lo | (hi << 16)).astype(jnp.int32)


def _mm1_body(x_ref, w_ref, dis_ref, o_ref, op_ref):
    h = jnp.dot(x_ref[...], w_ref[...], preferred_element_type=jnp.float32)
    g = h * dis_ref[...]
    o_ref[...] = g
    gp = _pack_g(g)
    op_ref[0] = gp
    op_ref[1] = gp


def _mid_body(p0_ref, p1_ref, g_ref, dis_ref, b_ref, w_ref, o_ref, op_ref):
    dis = dis_ref[...]
    y = (p0_ref[...] + p1_ref[...] + g_ref[...]) * dis + b_ref[...]
    y = jnp.maximum(y, 0.0)
    h2 = jnp.dot(y, w_ref[...], preferred_element_type=jnp.float32)
    g2 = h2 * dis
    o_ref[...] = g2
    gp = _pack_g(g2)
    op_ref[0] = gp
    op_ref[1] = gp


def _final_body(p0_ref, p1_ref, g_ref, dis_ref, b_ref, o_ref):
    o_ref[...] = ((p0_ref[...] + p1_ref[...] + g_ref[...]) * dis_ref[...]
                  + b_ref[...])


_MB = 1024
_GRID = (NP // _MB,)


def _blk(shape, imap):
    return pl.BlockSpec(shape, imap)


_FULL = _blk((_MB, D), lambda i: (i, 0))
_COL = _blk((_MB, 1), lambda i: (i, 0))
_ROW = _blk((1, D), lambda i: (0, 0))
_SQ = _blk((D, D), lambda i: (0, 0))


def _tc_prep(deg_partials):
    return pl.pallas_call(
        _prep_body,
        in_specs=[_blk((2 * NP // D, D), lambda: (0, 0))],
        out_specs=_blk((NP // D, D), lambda: (0, 0)),
        out_shape=jax.ShapeDtypeStruct((NP // D, D), jnp.float32),
    )(deg_partials)


_HALFW = _blk((2, _MB, DW), lambda i: (0, i, 0))


def _tc_mm1(x, w, dis):
    return pl.pallas_call(
        _mm1_body,
        grid=_GRID,
        in_specs=[_FULL, _SQ, _COL],
        out_specs=[_FULL, _HALFW],
        out_shape=[jax.ShapeDtypeStruct((NP, D), jnp.float32),
                   jax.ShapeDtypeStruct((2, NP, DW), jnp.int32)],
    )(x, w, dis)


def _tc_mid(p0, p1, g, dis, b, w):
    return pl.pallas_call(
        _mid_body,
        grid=_GRID,
        in_specs=[_FULL, _FULL, _FULL, _COL, _ROW, _SQ],
        out_specs=[_FULL, _HALFW],
        out_shape=[jax.ShapeDtypeStruct((NP, D), jnp.float32),
                   jax.ShapeDtypeStruct((2, NP, DW), jnp.int32)],
    )(p0, p1, g, dis, b, w)


def _tc_final(p0, p1, g, dis, b):
    return pl.pallas_call(
        _final_body,
        grid=_GRID,
        in_specs=[_FULL, _FULL, _FULL, _COL, _ROW],
        out_specs=_FULL,
        out_shape=jax.ShapeDtypeStruct((NP, D), jnp.float32),
    )(p0, p1, g, dis, b)


def kernel(x, edge_index, edge_weight, W1, b1, W2, b2):
    src = edge_index[0].astype(jnp.int32)
    dst = edge_index[1].astype(jnp.int32)
    ew = edge_weight.astype(jnp.float32)

    pad_e = EP - E
    src_p = jnp.concatenate([src, jnp.zeros((pad_e,), jnp.int32)])
    dst_p = jnp.concatenate([dst, jnp.zeros((pad_e,), jnp.int32)])
    ew_p = jnp.concatenate([ew, jnp.zeros((pad_e,), jnp.float32)])
    src3d = src_p.reshape(NW, NCHUNK, CH)
    dst3d = dst_p.reshape(NW, NCHUNK, CH)
    dst3d_deg = dst_p.reshape(NW, NCHD, CHD)
    x_p = jnp.concatenate(
        [x.astype(jnp.float32), jnp.zeros((NP - N, D), jnp.float32)])

    deg_partials = _deg_kernel(dst3d_deg, ew_p)
    dis2d = _tc_prep(deg_partials.reshape(2 * NP // D, D))
    dis = dis2d.reshape(NP, 1)

    g1, g1p = _tc_mm1(x_p, W1, dis)
    p1 = _layer_kernel(g1p, src3d, dst3d, ew_p)
    g2, g2p = _tc_mid(p1[0], p1[1], g1, dis, b1.reshape(1, D), W2)
    p2 = _layer_kernel(g2p, src3d, dst3d, ew_p)
    out = _tc_final(p2[0], p2[1], g2, dis, b2.reshape(1, D))
    return out[:N]
